# baseline (speedup 1.0000x reference)
import dataclasses
import functools

import jax
import jax.numpy as jnp
from jax import lax
from jax.experimental import pallas as pl
from jax.experimental.pallas import tpu as pltpu
from jax.experimental.pallas import tpu_sc as plsc

N = 10000
DH = 128
G = 64
NC = 2
NS = 16
NW = NC * NS
CH = 64
NBUF = 4
BLK = 32
NPAD = N + 16
R = 1000
DUMMY_DST = N

_mesh = plsc.VectorSubcoreMesh(core_axis_name="c", subcore_axis_name="s")

_sc_params = pltpu.CompilerParams()
if "needs_layout_passes" in pltpu.CompilerParams.__dataclass_fields__:
    _sc_params = dataclasses.replace(_sc_params, needs_layout_passes=False)

ROWS_SUB = (N // (NS * 8)) * 8
REM = N - ROWS_SUB * NS


def _num_chunks(E):
    raw = -(-E // (NW * CH))
    return -(-raw // NBUF) * NBUF


def _sc_edge_body(chunks, g_hbm, srcw_hbm, dstw_hbm,
                  out0_hbm, out1_hbm, sidx, didx, rows, acc,
                  sem0, sem1, sem2, sem3, semi):
    sems = (sem0, sem1, sem2, sem3)
    c = lax.axis_index("c")
    s = lax.axis_index("s")
    w = s * NC + c
    base = s * ROWS_SUB

    @pl.when(c == 0)
    def _():
        pltpu.sync_copy(g_hbm.at[pl.ds(base, ROWS_SUB)],
                        acc.at[pl.ds(base, ROWS_SUB)])

    @pl.when(c != 0)
    def _():
        @pl.loop(0, CH)
        def _(r):
            @pl.loop(0, DH // 16)
            def _(q):
                rows[0, r, pl.ds(q * 16, 16)] = jnp.zeros((16,), jnp.float32)

        @pl.loop(0, ROWS_SUB // CH)
        def _(t):
            pltpu.sync_copy(rows.at[0],
                            acc.at[pl.ds(base + t * CH, CH)])

        rem_rows = ROWS_SUB % CH
        if rem_rows:
            pltpu.sync_copy(
                rows.at[0, pl.ds(0, rem_rows)],
                acc.at[pl.ds(base + (ROWS_SUB // CH) * CH, rem_rows)])

    if REM > 0:
        @pl.when((s == 0) & (c == 0))
        def _():
            pltpu.sync_copy(g_hbm.at[pl.ds(ROWS_SUB * NS, REM)],
                            acc.at[pl.ds(ROWS_SUB * NS, REM)])

        @pl.when((s == 0) & (c != 0))
        def _():
            pltpu.sync_copy(rows.at[0, pl.ds(0, REM)],
                            acc.at[pl.ds(ROWS_SUB * NS, REM)])

    nblk = -(-chunks // BLK)
    nb0 = min(BLK, chunks)
    pltpu.async_copy(srcw_hbm.at[w, pl.ds(0, nb0)],
                     sidx.at[0, pl.ds(0, nb0)], semi)
    pltpu.async_copy(dstw_hbm.at[w, pl.ds(0, nb0)],
                     didx.at[0, pl.ds(0, nb0)], semi)
    plsc.subcore_barrier()

    for b in range(nblk):
        nb = min(BLK, chunks - b * BLK)
        cur = b % 2
        chain = b + 1 < nblk
        if b == 0:
            pltpu.make_async_copy(srcw_hbm.at[w, pl.ds(0, nb0)],
                                  sidx.at[0, pl.ds(0, nb0)], semi).wait()
            pltpu.make_async_copy(dstw_hbm.at[w, pl.ds(0, nb0)],
                                  didx.at[0, pl.ds(0, nb0)], semi).wait()
            for p in range(NBUF):
                pltpu.async_copy(g_hbm.at[sidx.at[0, p]], rows.at[p],
                                 sems[p])
        if chain:
            nb2 = min(BLK, chunks - (b + 1) * BLK)
            nxt = (b + 1) % 2
            pltpu.async_copy(srcw_hbm.at[w, pl.ds((b + 1) * BLK, nb2)],
                             sidx.at[nxt, pl.ds(0, nb2)], semi)
            pltpu.async_copy(dstw_hbm.at[w, pl.ds((b + 1) * BLK, nb2)],
                             didx.at[nxt, pl.ds(0, nb2)], semi)

        def _triple(k, guarded):
            for p in range(NBUF):
                pltpu.make_async_copy(g_hbm.at[sidx.at[cur, k + p]],
                                      rows.at[p], sems[p]).wait()
                pltpu.sync_copy(rows.at[p], acc.at[didx.at[cur, k + p]],
                                add=True)
                if not guarded:
                    pltpu.async_copy(g_hbm.at[sidx.at[cur, k + p + NBUF]],
                                     rows.at[p], sems[p])
                else:
                    @pl.when(k + p + NBUF < nb)
                    def _():
                        pltpu.async_copy(
                            g_hbm.at[sidx.at[cur, k + p + NBUF]],
                            rows.at[p], sems[p])

                    if chain:
                        @pl.when(k + p + NBUF >= nb)
                        def _():
                            pltpu.async_copy(
                                g_hbm.at[sidx.at[1 - cur, k + p + NBUF - nb]],
                                rows.at[p], sems[p])

        if chain:
            h = (nb // 2 // NBUF) * NBUF

            @pl.loop(0, h, step=NBUF)
            def _(k):
                _triple(k, guarded=False)

            pltpu.make_async_copy(srcw_hbm.at[w, pl.ds((b + 1) * BLK, nb2)],
                                  sidx.at[nxt, pl.ds(0, nb2)], semi).wait()
            pltpu.make_async_copy(dstw_hbm.at[w, pl.ds((b + 1) * BLK, nb2)],
                                  didx.at[nxt, pl.ds(0, nb2)], semi).wait()

            @pl.loop(h, nb, step=NBUF)
            def _(k):
                _triple(k, guarded=True)
        else:
            @pl.loop(0, nb, step=NBUF)
            def _(k):
                _triple(k, guarded=True)

    plsc.subcore_barrier()

    @pl.when(c == 0)
    def _():
        pltpu.sync_copy(acc.at[pl.ds(base, ROWS_SUB)],
                        out0_hbm.at[pl.ds(base, ROWS_SUB)])

    @pl.when(c != 0)
    def _():
        pltpu.sync_copy(acc.at[pl.ds(base, ROWS_SUB)],
                        out1_hbm.at[pl.ds(base, ROWS_SUB)])

    if REM > 0:
        @pl.when((s == 0) & (c == 0))
        def _():
            pltpu.sync_copy(acc.at[pl.ds(ROWS_SUB * NS, REM)],
                            out0_hbm.at[pl.ds(ROWS_SUB * NS, REM)])

        @pl.when((s == 0) & (c != 0))
        def _():
            pltpu.sync_copy(acc.at[pl.ds(ROWS_SUB * NS, REM)],
                            out1_hbm.at[pl.ds(ROWS_SUB * NS, REM)])


def _make_sc_edge(chunks):
    return pl.kernel(
        functools.partial(_sc_edge_body, chunks),
        mesh=_mesh,
        out_type=[jax.ShapeDtypeStruct((N, DH), jnp.float32),
                  jax.ShapeDtypeStruct((N, DH), jnp.float32)],
        scratch_types=[
            pltpu.VMEM((2, BLK, CH), jnp.int32),
            pltpu.VMEM((2, BLK, CH), jnp.int32),
            pltpu.VMEM((NBUF, CH, DH), jnp.float32),
            pltpu.VMEM_SHARED((NPAD, DH), jnp.float32),
            pltpu.SemaphoreType.DMA,
            pltpu.SemaphoreType.DMA,
            pltpu.SemaphoreType.DMA,
            pltpu.SemaphoreType.DMA,
            pltpu.SemaphoreType.DMA,
        ],
        compiler_params=_sc_params,
    )


def _sc_deg_body(chunks, dstw_hbm, degp_hbm, didx, dloc):
    c = lax.axis_index("c")
    s = lax.axis_index("s")
    w = s * NC + c
    pltpu.sync_copy(dstw_hbm.at[w], didx)

    @pl.loop(0, NPAD // 16)
    def _(i):
        dloc[pl.ds(i * 16, 16)] = jnp.zeros((16,), jnp.float32)

    ones = jnp.ones((16,), jnp.float32)

    @pl.loop(0, chunks)
    def _(j):
        @pl.loop(0, CH // 16)
        def _(k):
            idx = didx[j, pl.ds(k * 16, 16)]
            plsc.addupdate_scatter(dloc, [idx], ones)

    pltpu.sync_copy(dloc, degp_hbm.at[w])


def _make_sc_deg(chunks):
    return pl.kernel(
        functools.partial(_sc_deg_body, chunks),
        mesh=_mesh,
        out_type=jax.ShapeDtypeStruct((NW, NPAD), jnp.float32),
        scratch_types=[
            pltpu.VMEM((chunks, CH), jnp.int32),
            pltpu.VMEM((NPAD,), jnp.float32),
        ],
        compiler_params=_sc_params,
    )


def _tc_dinv_body(degp_ref, o_ref):
    deg = jnp.sum(degp_ref[...], axis=0) + 1.0
    o_ref[...] = (1.0 / jnp.sqrt(deg))[:, None]


def _tc_first_body(x_ref, dinv_ref, w_ref, g_ref):
    h = jnp.dot(x_ref[...], w_ref[...], preferred_element_type=jnp.float32)
    g_ref[...] = dinv_ref[...] * h


def _tc_mid_body(a0_ref, a1_ref, dinv_ref, b_ref, w_ref, g_ref, *, act):
    dinv = dinv_ref[...]
    a = dinv * (a0_ref[...] + a1_ref[...]) + b_ref[...]
    if act:
        a = jnp.maximum(a, 0.0)
    h = jnp.dot(a, w_ref[...], preferred_element_type=jnp.float32)
    g_ref[...] = dinv * h


def _tc_pool_body(a0_ref, a1_ref, dinv_ref, b_ref, batch_ref, wc_ref, bc_ref,
                  o_ref, psum_ref, pcnt_ref):
    i = pl.program_id(0)

    @pl.when(i == 0)
    def _():
        psum_ref[...] = jnp.zeros_like(psum_ref)
        pcnt_ref[...] = jnp.zeros_like(pcnt_ref)

    h = dinv_ref[...] * (a0_ref[...] + a1_ref[...]) + b_ref[...]
    b = batch_ref[0, 0, :]
    gid = lax.broadcasted_iota(jnp.int32, (G, R), 0)
    m = (gid == b[None, :]).astype(jnp.float32)
    psum_ref[...] += jnp.dot(m, h, preferred_element_type=jnp.float32)
    pcnt_ref[...] += jnp.sum(m, axis=1, keepdims=True)

    @pl.when(i == pl.num_programs(0) - 1)
    def _():
        pooled = psum_ref[...] / jnp.maximum(pcnt_ref[...], 1.0)
        o_ref[...] = (jnp.dot(pooled, wc_ref[...],
                              preferred_element_type=jnp.float32)
                      + bc_ref[...])


def kernel(x, edge_index, batch, W1, b1, W2, b2, W3, b3, Wc, bc):
    E = edge_index.shape[1]
    DOUT = Wc.shape[1]
    chunks = _num_chunks(E)

    src = edge_index[0]
    dst = edge_index[1]
    r = (-E) % NW
    if r:
        src = jnp.concatenate([src, jnp.zeros((r,), jnp.int32)])
        dst = jnp.concatenate([dst, jnp.full((r,), DUMMY_DST, jnp.int32)])
    per_w = (E + r) // NW
    k = chunks * CH - per_w
    src2 = src.reshape(NW, per_w)
    dst2 = dst.reshape(NW, per_w)
    if k:
        dpad = jnp.broadcast_to(
            DUMMY_DST + (jnp.arange(k, dtype=jnp.int32) % (NPAD - N)),
            (NW, k))
        src2 = jnp.concatenate([src2, jnp.zeros((NW, k), jnp.int32)], axis=1)
        dst2 = jnp.concatenate([dst2, dpad], axis=1)
    srcw = src2.reshape(NW, chunks, CH)
    dstw = dst2.reshape(NW, chunks, CH)
    sc_deg = _make_sc_deg(chunks)
    sc_edge = _make_sc_edge(chunks)

    degp = sc_deg(dstw)
    dinv = pl.pallas_call(
        _tc_dinv_body,
        out_shape=jax.ShapeDtypeStruct((NPAD, 1), jnp.float32),
    )(degp)

    grid = N // R
    tc_first = pl.pallas_call(
        _tc_first_body,
        grid=(grid,),
        in_specs=[
            pl.BlockSpec((R, DH), lambda i: (i, 0)),
            pl.BlockSpec((R, 1), lambda i: (i, 0)),
            pl.BlockSpec((DH, DH), lambda i: (0, 0)),
        ],
        out_specs=pl.BlockSpec((R, DH), lambda i: (i, 0)),
        out_shape=jax.ShapeDtypeStruct((N, DH), jnp.float32),
    )

    def tc_mid(act):
        return pl.pallas_call(
            functools.partial(_tc_mid_body, act=act),
            grid=(grid,),
            in_specs=[
                pl.BlockSpec((R, DH), lambda i: (i, 0)),
                pl.BlockSpec((R, DH), lambda i: (i, 0)),
                pl.BlockSpec((R, 1), lambda i: (i, 0)),
                pl.BlockSpec((1, DH), lambda i: (0, 0)),
                pl.BlockSpec((DH, DH), lambda i: (0, 0)),
            ],
            out_specs=pl.BlockSpec((R, DH), lambda i: (i, 0)),
            out_shape=jax.ShapeDtypeStruct((N, DH), jnp.float32),
        )

    tc_pool = pl.pallas_call(
        _tc_pool_body,
        grid=(grid,),
        in_specs=[
            pl.BlockSpec((R, DH), lambda i: (i, 0)),
            pl.BlockSpec((R, DH), lambda i: (i, 0)),
            pl.BlockSpec((R, 1), lambda i: (i, 0)),
            pl.BlockSpec((1, DH), lambda i: (0, 0)),
            pl.BlockSpec((1, 1, R), lambda i: (i, 0, 0)),
            pl.BlockSpec((DH, DOUT), lambda i: (0, 0)),
            pl.BlockSpec((1, DOUT), lambda i: (0, 0)),
        ],
        out_specs=pl.BlockSpec((G, DOUT), lambda i: (0, 0)),
        out_shape=jax.ShapeDtypeStruct((G, DOUT), jnp.float32),
        scratch_shapes=[
            pltpu.VMEM((G, DH), jnp.float32),
            pltpu.VMEM((G, DH), jnp.float32),
        ],
    )

    batch3 = batch.reshape(grid, 1, R)

    g1 = tc_first(x, dinv, W1)
    a0, a1 = sc_edge(g1, srcw, dstw)
    g2 = tc_mid(True)(a0, a1, dinv, b1.reshape(1, DH), W2)
    a0, a1 = sc_edge(g2, srcw, dstw)
    g3 = tc_mid(True)(a0, a1, dinv, b2.reshape(1, DH), W3)
    a0, a1 = sc_edge(g3, srcw, dstw)
    return tc_pool(a0, a1, dinv, b3.reshape(1, DH), batch3, Wc,
                   bc.reshape(1, DOUT))


# speedup vs baseline: 1.8758x; 1.8758x over previous
import dataclasses
import functools

import jax
import jax.numpy as jnp
from jax import lax
from jax.experimental import pallas as pl
from jax.experimental.pallas import tpu as pltpu
from jax.experimental.pallas import tpu_sc as plsc

N = 10000
DH = 128
G = 64
NC = 2
NS = 16
NW = NC * NS
CH = 96
NBUF = 3
BLK = 24
NPAD = N + 16
R = 1000
DUMMY_DST = N

_mesh = plsc.VectorSubcoreMesh(core_axis_name="c", subcore_axis_name="s")

_sc_params = pltpu.CompilerParams()
if "needs_layout_passes" in pltpu.CompilerParams.__dataclass_fields__:
    _sc_params = dataclasses.replace(_sc_params, needs_layout_passes=False)

ROWS_SUB = (N // (NS * 8)) * 8
REM = N - ROWS_SUB * NS


def _num_chunks(E):
    raw = -(-E // (NW * CH))
    return -(-raw // NBUF) * NBUF


def _sc_edge_body(chunks, g_hbm, srcw_hbm, dstw_hbm,
                  out0_hbm, out1_hbm, sidx, didx, rows, acc,
                  sem0, sem1, sem2, semi):
    sems = (sem0, sem1, sem2)
    c = lax.axis_index("c")
    s = lax.axis_index("s")
    w = s * NC + c
    base = s * ROWS_SUB

    @pl.when(c == 0)
    def _():
        pltpu.sync_copy(g_hbm.at[pl.ds(base, ROWS_SUB)],
                        acc.at[pl.ds(base, ROWS_SUB)])

    @pl.when(c != 0)
    def _():
        @pl.loop(0, CH)
        def _(r):
            @pl.loop(0, DH // 16)
            def _(q):
                rows[0, r, pl.ds(q * 16, 16)] = jnp.zeros((16,), jnp.float32)

        @pl.loop(0, ROWS_SUB // CH)
        def _(t):
            pltpu.sync_copy(rows.at[0],
                            acc.at[pl.ds(base + t * CH, CH)])

        rem_rows = ROWS_SUB % CH
        if rem_rows:
            pltpu.sync_copy(
                rows.at[0, pl.ds(0, rem_rows)],
                acc.at[pl.ds(base + (ROWS_SUB // CH) * CH, rem_rows)])

    if REM > 0:
        @pl.when((s == 0) & (c == 0))
        def _():
            pltpu.sync_copy(g_hbm.at[pl.ds(ROWS_SUB * NS, REM)],
                            acc.at[pl.ds(ROWS_SUB * NS, REM)])

        @pl.when((s == 0) & (c != 0))
        def _():
            pltpu.sync_copy(rows.at[0, pl.ds(0, REM)],
                            acc.at[pl.ds(ROWS_SUB * NS, REM)])

    nblk = -(-chunks // BLK)
    nb0 = min(BLK, chunks)
    pltpu.async_copy(srcw_hbm.at[w, pl.ds(0, nb0)],
                     sidx.at[0, pl.ds(0, nb0)], semi)
    pltpu.async_copy(dstw_hbm.at[w, pl.ds(0, nb0)],
                     didx.at[0, pl.ds(0, nb0)], semi)
    plsc.subcore_barrier()

    for b in range(nblk):
        nb = min(BLK, chunks - b * BLK)
        cur = b % 2
        chain = b + 1 < nblk
        if b == 0:
            pltpu.make_async_copy(srcw_hbm.at[w, pl.ds(0, nb0)],
                                  sidx.at[0, pl.ds(0, nb0)], semi).wait()
            pltpu.make_async_copy(dstw_hbm.at[w, pl.ds(0, nb0)],
                                  didx.at[0, pl.ds(0, nb0)], semi).wait()
            for p in range(NBUF):
                pltpu.async_copy(g_hbm.at[sidx.at[0, p]], rows.at[p],
                                 sems[p])
        if chain:
            nb2 = min(BLK, chunks - (b + 1) * BLK)
            nxt = (b + 1) % 2
            pltpu.async_copy(srcw_hbm.at[w, pl.ds((b + 1) * BLK, nb2)],
                             sidx.at[nxt, pl.ds(0, nb2)], semi)
            pltpu.async_copy(dstw_hbm.at[w, pl.ds((b + 1) * BLK, nb2)],
                             didx.at[nxt, pl.ds(0, nb2)], semi)

        def _triple(k, guarded):
            for p in range(NBUF):
                pltpu.make_async_copy(g_hbm.at[sidx.at[cur, k + p]],
                                      rows.at[p], sems[p]).wait()
                pltpu.sync_copy(rows.at[p], acc.at[didx.at[cur, k + p]],
                                add=True)
                if not guarded:
                    pltpu.async_copy(g_hbm.at[sidx.at[cur, k + p + NBUF]],
                                     rows.at[p], sems[p])
                else:
                    @pl.when(k + p + NBUF < nb)
                    def _():
                        pltpu.async_copy(
                            g_hbm.at[sidx.at[cur, k + p + NBUF]],
                            rows.at[p], sems[p])

                    if chain:
                        @pl.when(k + p + NBUF >= nb)
                        def _():
                            pltpu.async_copy(
                                g_hbm.at[sidx.at[1 - cur, k + p + NBUF - nb]],
                                rows.at[p], sems[p])

        if chain:
            h = (nb // 2 // NBUF) * NBUF

            @pl.loop(0, h, step=NBUF)
            def _(k):
                _triple(k, guarded=False)

            pltpu.make_async_copy(srcw_hbm.at[w, pl.ds((b + 1) * BLK, nb2)],
                                  sidx.at[nxt, pl.ds(0, nb2)], semi).wait()
            pltpu.make_async_copy(dstw_hbm.at[w, pl.ds((b + 1) * BLK, nb2)],
                                  didx.at[nxt, pl.ds(0, nb2)], semi).wait()

            @pl.loop(h, nb, step=NBUF)
            def _(k):
                _triple(k, guarded=True)
        else:
            @pl.loop(0, nb, step=NBUF)
            def _(k):
                _triple(k, guarded=True)

    plsc.subcore_barrier()

    @pl.when(c == 0)
    def _():
        pltpu.sync_copy(acc.at[pl.ds(base, ROWS_SUB)],
                        out0_hbm.at[pl.ds(base, ROWS_SUB)])

    @pl.when(c != 0)
    def _():
        pltpu.sync_copy(acc.at[pl.ds(base, ROWS_SUB)],
                        out1_hbm.at[pl.ds(base, ROWS_SUB)])

    if REM > 0:
        @pl.when((s == 0) & (c == 0))
        def _():
            pltpu.sync_copy(acc.at[pl.ds(ROWS_SUB * NS, REM)],
                            out0_hbm.at[pl.ds(ROWS_SUB * NS, REM)])

        @pl.when((s == 0) & (c != 0))
        def _():
            pltpu.sync_copy(acc.at[pl.ds(ROWS_SUB * NS, REM)],
                            out1_hbm.at[pl.ds(ROWS_SUB * NS, REM)])


def _make_sc_edge(chunks):
    return pl.kernel(
        functools.partial(_sc_edge_body, chunks),
        mesh=_mesh,
        out_type=[jax.ShapeDtypeStruct((N, DH), jnp.float32),
                  jax.ShapeDtypeStruct((N, DH), jnp.float32)],
        scratch_types=[
            pltpu.VMEM((2, BLK, CH), jnp.int32),
            pltpu.VMEM((2, BLK, CH), jnp.int32),
            pltpu.VMEM((NBUF, CH, DH), jnp.float32),
            pltpu.VMEM_SHARED((NPAD, DH), jnp.float32),
            pltpu.SemaphoreType.DMA,
            pltpu.SemaphoreType.DMA,
            pltpu.SemaphoreType.DMA,
            pltpu.SemaphoreType.DMA,
        ],
        compiler_params=_sc_params,
    )


def _sc_deg_body(chunks, dstw_hbm, degp_hbm, didx, dloc):
    c = lax.axis_index("c")
    s = lax.axis_index("s")
    w = s * NC + c
    pltpu.sync_copy(dstw_hbm.at[w], didx)

    @pl.loop(0, NPAD // 16)
    def _(i):
        dloc[pl.ds(i * 16, 16)] = jnp.zeros((16,), jnp.float32)

    ones = jnp.ones((16,), jnp.float32)

    @pl.loop(0, chunks)
    def _(j):
        @pl.loop(0, CH // 16)
        def _(k):
            idx = didx[j, pl.ds(k * 16, 16)]
            plsc.addupdate_scatter(dloc, [idx], ones)

    pltpu.sync_copy(dloc, degp_hbm.at[w])


def _make_sc_deg(chunks):
    return pl.kernel(
        functools.partial(_sc_deg_body, chunks),
        mesh=_mesh,
        out_type=jax.ShapeDtypeStruct((NW, NPAD), jnp.float32),
        scratch_types=[
            pltpu.VMEM((chunks, CH), jnp.int32),
            pltpu.VMEM((NPAD,), jnp.float32),
        ],
        compiler_params=_sc_params,
    )


def _tc_dinv_body(degp_ref, o_ref):
    deg = jnp.sum(degp_ref[...], axis=0) + 1.0
    o_ref[...] = (1.0 / jnp.sqrt(deg))[:, None]


def _tc_first_body(x_ref, dinv_ref, w_ref, g_ref):
    h = jnp.dot(x_ref[...], w_ref[...], preferred_element_type=jnp.float32)
    g_ref[...] = dinv_ref[...] * h


def _tc_mid_body(a0_ref, a1_ref, dinv_ref, b_ref, w_ref, g_ref, *, act):
    dinv = dinv_ref[...]
    a = dinv * (a0_ref[...] + a1_ref[...]) + b_ref[...]
    if act:
        a = jnp.maximum(a, 0.0)
    h = jnp.dot(a, w_ref[...], preferred_element_type=jnp.float32)
    g_ref[...] = dinv * h


def _tc_pool_body(a0_ref, a1_ref, dinv_ref, b_ref, batch_ref, wc_ref, bc_ref,
                  o_ref, psum_ref, pcnt_ref):
    i = pl.program_id(0)

    @pl.when(i == 0)
    def _():
        psum_ref[...] = jnp.zeros_like(psum_ref)
        pcnt_ref[...] = jnp.zeros_like(pcnt_ref)

    h = dinv_ref[...] * (a0_ref[...] + a1_ref[...]) + b_ref[...]
    b = batch_ref[0, 0, :]
    gid = lax.broadcasted_iota(jnp.int32, (G, R), 0)
    m = (gid == b[None, :]).astype(jnp.float32)
    psum_ref[...] += jnp.dot(m, h, preferred_element_type=jnp.float32)
    pcnt_ref[...] += jnp.sum(m, axis=1, keepdims=True)

    @pl.when(i == pl.num_programs(0) - 1)
    def _():
        pooled = psum_ref[...] / jnp.maximum(pcnt_ref[...], 1.0)
        o_ref[...] = (jnp.dot(pooled, wc_ref[...],
                              preferred_element_type=jnp.float32)
                      + bc_ref[...])


def kernel(x, edge_index, batch, W1, b1, W2, b2, W3, b3, Wc, bc):
    E = edge_index.shape[1]
    DOUT = Wc.shape[1]
    chunks = _num_chunks(E)

    src = edge_index[0]
    dst = edge_index[1]
    r = (-E) % NW
    if r:
        src = jnp.concatenate([src, jnp.zeros((r,), jnp.int32)])
        dst = jnp.concatenate([dst, jnp.full((r,), DUMMY_DST, jnp.int32)])
    per_w = (E + r) // NW
    k = chunks * CH - per_w
    src2 = src.reshape(NW, per_w)
    dst2 = dst.reshape(NW, per_w)
    if k:
        dpad = jnp.broadcast_to(
            DUMMY_DST + (jnp.arange(k, dtype=jnp.int32) % (NPAD - N)),
            (NW, k))
        src2 = jnp.concatenate([src2, jnp.zeros((NW, k), jnp.int32)], axis=1)
        dst2 = jnp.concatenate([dst2, dpad], axis=1)
    srcw = src2.reshape(NW, chunks, CH)
    dstw = dst2.reshape(NW, chunks, CH)
    sc_deg = _make_sc_deg(chunks)
    sc_edge = _make_sc_edge(chunks)

    degp = sc_deg(dstw)
    dinv = pl.pallas_call(
        _tc_dinv_body,
        out_shape=jax.ShapeDtypeStruct((NPAD, 1), jnp.float32),
    )(degp)

    grid = N // R
    tc_first = pl.pallas_call(
        _tc_first_body,
        grid=(grid,),
        in_specs=[
            pl.BlockSpec((R, DH), lambda i: (i, 0)),
            pl.BlockSpec((R, 1), lambda i: (i, 0)),
            pl.BlockSpec((DH, DH), lambda i: (0, 0)),
        ],
        out_specs=pl.BlockSpec((R, DH), lambda i: (i, 0)),
        out_shape=jax.ShapeDtypeStruct((N, DH), jnp.float32),
    )

    def tc_mid(act):
        return pl.pallas_call(
            functools.partial(_tc_mid_body, act=act),
            grid=(grid,),
            in_specs=[
                pl.BlockSpec((R, DH), lambda i: (i, 0)),
                pl.BlockSpec((R, DH), lambda i: (i, 0)),
                pl.BlockSpec((R, 1), lambda i: (i, 0)),
                pl.BlockSpec((1, DH), lambda i: (0, 0)),
                pl.BlockSpec((DH, DH), lambda i: (0, 0)),
            ],
            out_specs=pl.BlockSpec((R, DH), lambda i: (i, 0)),
            out_shape=jax.ShapeDtypeStruct((N, DH), jnp.float32),
        )

    tc_pool = pl.pallas_call(
        _tc_pool_body,
        grid=(grid,),
        in_specs=[
            pl.BlockSpec((R, DH), lambda i: (i, 0)),
            pl.BlockSpec((R, DH), lambda i: (i, 0)),
            pl.BlockSpec((R, 1), lambda i: (i, 0)),
            pl.BlockSpec((1, DH), lambda i: (0, 0)),
            pl.BlockSpec((1, 1, R), lambda i: (i, 0, 0)),
            pl.BlockSpec((DH, DOUT), lambda i: (0, 0)),
            pl.BlockSpec((1, DOUT), lambda i: (0, 0)),
        ],
        out_specs=pl.BlockSpec((G, DOUT), lambda i: (0, 0)),
        out_shape=jax.ShapeDtypeStruct((G, DOUT), jnp.float32),
        scratch_shapes=[
            pltpu.VMEM((G, DH), jnp.float32),
            pltpu.VMEM((G, DH), jnp.float32),
        ],
    )

    batch3 = batch.reshape(grid, 1, R)

    g1 = tc_first(x, dinv, W1)
    a0, a1 = sc_edge(g1, srcw, dstw)
    g2 = tc_mid(True)(a0, a1, dinv, b1.reshape(1, DH), W2)
    a0, a1 = sc_edge(g2, srcw, dstw)
    g3 = tc_mid(True)(a0, a1, dinv, b2.reshape(1, DH), W3)
    a0, a1 = sc_edge(g3, srcw, dstw)
    return tc_pool(a0, a1, dinv, b3.reshape(1, DH), batch3, Wc,
                   bc.reshape(1, DOUT))
